# baseline (speedup 1.0000x reference)
import functools

import jax
import jax.numpy as jnp
from jax import lax
from jax.experimental import pallas as pl
from jax.experimental.pallas import tpu as pltpu
from jax.experimental.pallas import tpu_sc as plsc

BATCH = 64
PLEN = 50
D = 24576
NC, NS = 2, 16
NW = NC * NS
NSLICE = 16
DC = D // NSLICE
BH = BATCH // 2
GRP = 16


def kernel(prompt, embedding):
    idx = prompt.T
    mesh = plsc.VectorSubcoreMesh(core_axis_name="c", subcore_axis_name="s")

    @functools.partial(
        pl.kernel,
        out_type=jax.ShapeDtypeStruct((PLEN, BATCH, D), jnp.float32),
        mesh=mesh,
        compiler_params=pltpu.CompilerParams(skip_device_barrier=True),
        scratch_types=[
            pltpu.VMEM((PLEN, BATCH), jnp.int32),
            pltpu.VMEM((PLEN, DC), jnp.float32),
            pltpu.SemaphoreType.DMA,
        ],
    )
    def run(emb_hbm, idx_hbm, out_hbm, idx_v, cache, sem):
        wid = lax.axis_index("s") * NC + lax.axis_index("c")
        sl = wid % NSLICE
        half = wid // NSLICE
        d0 = sl * DC
        pltpu.sync_copy(idx_hbm, idx_v)
        pltpu.sync_copy(emb_hbm.at[:, pl.ds(d0, DC)], cache)

        def piece(row, p, b):
            return pltpu.make_async_copy(
                cache.at[row], out_hbm.at[p, b, pl.ds(d0, DC)], sem)

        @pl.loop(0, PLEN * (BH // GRP))
        def _(g):
            p = g // (BH // GRP)
            mg = g - p * (BH // GRP)
            b0 = half * BH + mg * GRP
            v = idx_v[p, pl.ds(b0, GRP)]
            for k in range(GRP):
                piece(v[k], p, b0 + k).start()

                @pl.when(g > 0)
                def _():
                    piece(jnp.int32(0), 0, 0).wait()

        for _ in range(GRP):
            piece(jnp.int32(0), 0, 0).wait()

    return run(embedding, idx).transpose(1, 0, 2)


# speedup vs baseline: 1.0021x; 1.0021x over previous
import functools

import jax
import jax.numpy as jnp
from jax import lax
from jax.experimental import pallas as pl
from jax.experimental.pallas import tpu as pltpu
from jax.experimental.pallas import tpu_sc as plsc

BATCH = 64
PLEN = 50
D = 24576
NC, NS = 2, 16
NW = NC * NS
NSLICE = 16
DC = D // NSLICE
BH = BATCH // 2
GRP = 16


def kernel(prompt, embedding):
    idx = prompt.T
    mesh = plsc.VectorSubcoreMesh(core_axis_name="c", subcore_axis_name="s")

    @functools.partial(
        pl.kernel,
        out_type=jax.ShapeDtypeStruct((PLEN, BATCH, D), jnp.float32),
        mesh=mesh,
        scratch_types=[
            pltpu.VMEM((PLEN, BATCH), jnp.int32),
            pltpu.VMEM((PLEN, DC), jnp.float32),
            pltpu.SemaphoreType.DMA,
        ],
    )
    def run(emb_hbm, idx_hbm, out_hbm, idx_v, cache, sem):
        wid = lax.axis_index("s") * NC + lax.axis_index("c")
        sl = wid % NSLICE
        half = wid // NSLICE
        d0 = sl * DC
        pltpu.sync_copy(idx_hbm, idx_v)
        pltpu.sync_copy(emb_hbm.at[:, pl.ds(d0, DC)], cache)

        def piece(row, p, b):
            return pltpu.make_async_copy(
                cache.at[row], out_hbm.at[p, b, pl.ds(d0, DC)], sem)

        @pl.loop(0, PLEN * (BH // GRP))
        def _(g):
            p = g // (BH // GRP)
            mg = g - p * (BH // GRP)
            b0 = half * BH + mg * GRP
            v = idx_v[p, pl.ds(b0, GRP)]
            for k in range(GRP):
                piece(v[k], p, b0 + k).start()

                @pl.when(g > 0)
                def _():
                    piece(jnp.int32(0), 0, 0).wait()

        for _ in range(GRP):
            piece(jnp.int32(0), 0, 0).wait()

    return run(embedding, idx).transpose(1, 0, 2)
